# baseline (speedup 1.0000x reference)
import functools

import jax
import jax.numpy as jnp
from jax import lax
from jax.experimental import pallas as pl
from jax.experimental.pallas import tpu as pltpu, tpu_sc as plsc


_C = 128


def _sc_call(x):
    batch, s, _ = x.shape
    nb = s // _C
    units = batch * nb
    info = plsc.get_sparse_core_info()
    nw = info.num_cores * info.num_subcores
    per_w = units // nw

    mesh = plsc.VectorSubcoreMesh(core_axis_name="c", subcore_axis_name="s")

    @functools.partial(
        pl.kernel,
        mesh=mesh,
        out_type=jax.ShapeDtypeStruct((batch, s, s), jnp.float32),
        scratch_types=[
            pltpu.VMEM((_C, _C), jnp.float32),
            pltpu.VMEM((_C, _C), jnp.float32),
            pltpu.SemaphoreType.DMA,
        ],
    )
    def run(x_hbm, out_hbm, ones_t, stage_t, sem):
        wid = lax.axis_index("s") * info.num_cores + lax.axis_index("c")

        one = jnp.full((16,), 1.0, jnp.float32)

        @pl.loop(0, _C)
        def _(r):
            for g in range(_C // 16):
                ones_t[r, pl.ds(g * 16, 16)] = one

        for u in range(per_w):
            unit = u * nw + wid
            b = lax.div(unit, nb)
            band = lax.rem(unit, nb)
            rows = pl.ds(band * _C, _C)
            for k in range(nb):
                sl = slice(k * _C, (k + 1) * _C)

                @pl.when(k < band)
                def _(sl=sl, rows=rows, b=b):
                    pltpu.async_copy(
                        x_hbm.at[b, rows, sl], stage_t, sem
                    ).wait()
                    pltpu.sync_copy(stage_t, out_hbm.at[b, rows, sl])

                @pl.when(k == band)
                def _(sl=sl, rows=rows, b=b):
                    pltpu.async_copy(
                        x_hbm.at[b, rows, sl], stage_t, sem
                    ).wait()

                    @pl.loop(0, _C)
                    def _(r):
                        for g in range(_C // 16):
                            col = lax.iota(jnp.int32, 16) + g * 16
                            cur = stage_t[r, pl.ds(g * 16, 16)]
                            stage_t[r, pl.ds(g * 16, 16)] = jnp.where(
                                col > r, jnp.float32(1.0), cur
                            )

                    pltpu.sync_copy(stage_t, out_hbm.at[b, rows, sl])

                @pl.when(k > band)
                def _(sl=sl, rows=rows, b=b):
                    pltpu.sync_copy(ones_t, out_hbm.at[b, rows, sl])

    return run


def kernel(x):
    return _sc_call(x)(x)


# speedup vs baseline: 2.8619x; 2.8619x over previous
import jax
import jax.numpy as jnp
from jax.experimental import pallas as pl
import jax.experimental.pallas.tpu as pltpu


_BAND = 256
_SLOTS = 4
_H = _BAND // 2


def _diag_squares(r0, c0, m):
    if m <= 128:
        return [(r0, c0, m, m)]
    h = m // 2
    return ([(r0 + h, c0, h, h)]
            + _diag_squares(r0, c0, h)
            + _diag_squares(r0 + h, c0 + h, h))


def _diag_quadrant_copies(x_ref, buf, sem_r, band, slot, k):
    base = k * _BAND
    out = []
    for r0, c0, hr, hc in _diag_squares(0, 0, _BAND):
        out.append(pltpu.make_async_copy(
            x_ref.at[:, pl.ds(band * _BAND + r0, hr),
                     slice(base + c0, base + c0 + hc)],
            buf.at[slot, :, slice(r0, r0 + hr),
                   slice(base + c0, base + c0 + hc)],
            sem_r.at[slot],
        ))
    return out


def _read_band(x_ref, buf, sem_r, band, slot, nc):
    for k in range(nc):
        sl = slice(k * _BAND, (k + 1) * _BAND)

        @pl.when(k < band)
        def _(sl=sl):
            pltpu.make_async_copy(
                x_ref.at[:, pl.ds(band * _BAND, _BAND), sl],
                buf.at[slot, :, :, sl],
                sem_r.at[slot],
            ).start()

        @pl.when(k == band)
        def _(k=k):
            for cp in _diag_quadrant_copies(x_ref, buf, sem_r, band, slot, k):
                cp.start()


def _wait_band(x_ref, buf, sem_r, band, slot, nc):
    for k in range(nc):
        sl = slice(k * _BAND, (k + 1) * _BAND)

        @pl.when(k < band)
        def _(sl=sl):
            pltpu.make_async_copy(
                x_ref.at[:, pl.ds(band * _BAND, _BAND), sl],
                buf.at[slot, :, :, sl],
                sem_r.at[slot],
            ).wait()

        @pl.when(k == band)
        def _(k=k):
            for cp in _diag_quadrant_copies(x_ref, buf, sem_r, band, slot, k):
                cp.wait()


def _write_copy(o_ref, buf, sem_w, band, slot):
    return pltpu.make_async_copy(
        buf.at[slot],
        o_ref.at[:, pl.ds(band * _BAND, _BAND), :],
        sem_w.at[slot],
    )


def _body(x_ref, o_ref, buf, sem_r, sem_w):
    i = pl.program_id(0)
    n = pl.num_programs(0)
    s = x_ref.shape[2]
    nc = s // _BAND
    slot = jax.lax.rem(i, _SLOTS)
    nxt = jax.lax.rem(i + 1, _SLOTS)

    @pl.when(i == 0)
    def _():
        _read_band(x_ref, buf, sem_r, 0, jnp.int32(0), nc)

    @pl.when(jnp.logical_and(i + 1 < n, i >= _SLOTS - 1))
    def _():
        _write_copy(o_ref, buf, sem_w, i - (_SLOTS - 1), nxt).wait()

    @pl.when(i + 1 < n)
    def _():
        _read_band(x_ref, buf, sem_r, i + 1, nxt, nc)

    for k in range(nc):
        sl = slice(k * _BAND, (k + 1) * _BAND)

        @pl.when(k > i)
        def _(sl=sl):
            buf[slot, :, :, sl] = jnp.ones(
                (buf.shape[1], _BAND, _BAND), jnp.float32
            )

    _wait_band(x_ref, buf, sem_r, i, slot, nc)

    for k in range(nc):
        sl = slice(k * _BAND, (k + 1) * _BAND)

        @pl.when(k == i)
        def _(sl=sl):
            r = jax.lax.broadcasted_iota(jnp.int32, (1, _BAND, _BAND), 1)
            c = jax.lax.broadcasted_iota(jnp.int32, (1, _BAND, _BAND), 2)
            buf[slot, :, :, sl] = jnp.where(
                c > r, jnp.float32(1.0), buf[slot, :, :, sl]
            )

    _write_copy(o_ref, buf, sem_w, i, slot).start()

    n_static = x_ref.shape[1] // _BAND

    @pl.when(i == n - 1)
    def _():
        for band in range(max(0, n_static - _SLOTS), n_static):
            _write_copy(o_ref, buf, sem_w, band, band % _SLOTS).wait()


def kernel(x):
    batch, s, _ = x.shape
    n = s // _BAND
    return pl.pallas_call(
        _body,
        grid=(n,),
        in_specs=[pl.BlockSpec(memory_space=pltpu.MemorySpace.HBM)],
        out_specs=pl.BlockSpec(memory_space=pltpu.MemorySpace.HBM),
        out_shape=jax.ShapeDtypeStruct(x.shape, x.dtype),
        scratch_shapes=[
            pltpu.VMEM((_SLOTS, batch, _BAND, s), jnp.float32),
            pltpu.SemaphoreType.DMA((_SLOTS,)),
            pltpu.SemaphoreType.DMA((_SLOTS,)),
        ],
    )(x)


# speedup vs baseline: 2.8794x; 1.0061x over previous
import jax
import jax.numpy as jnp
from jax.experimental import pallas as pl
import jax.experimental.pallas.tpu as pltpu


_BAND = 256
_SLOTS = 5
_H = _BAND // 2


def _diag_squares(r0, c0, m):
    if m <= 128:
        return [(r0, c0, m, m)]
    h = m // 2
    return ([(r0 + h, c0, h, h)]
            + _diag_squares(r0, c0, h)
            + _diag_squares(r0 + h, c0 + h, h))


def _diag_quadrant_copies(x_ref, buf, sem_r, band, slot, k):
    base = k * _BAND
    out = []
    for r0, c0, hr, hc in _diag_squares(0, 0, _BAND):
        out.append(pltpu.make_async_copy(
            x_ref.at[:, pl.ds(band * _BAND + r0, hr),
                     slice(base + c0, base + c0 + hc)],
            buf.at[slot, :, slice(r0, r0 + hr),
                   slice(base + c0, base + c0 + hc)],
            sem_r.at[slot],
        ))
    return out


def _read_band(x_ref, buf, sem_r, band, slot, nc):
    for k in range(nc):
        sl = slice(k * _BAND, (k + 1) * _BAND)

        @pl.when(k < band)
        def _(sl=sl):
            pltpu.make_async_copy(
                x_ref.at[:, pl.ds(band * _BAND, _BAND), sl],
                buf.at[slot, :, :, sl],
                sem_r.at[slot],
            ).start()

        @pl.when(k == band)
        def _(k=k):
            for cp in _diag_quadrant_copies(x_ref, buf, sem_r, band, slot, k):
                cp.start()


def _wait_band(x_ref, buf, sem_r, band, slot, nc):
    for k in range(nc):
        sl = slice(k * _BAND, (k + 1) * _BAND)

        @pl.when(k < band)
        def _(sl=sl):
            pltpu.make_async_copy(
                x_ref.at[:, pl.ds(band * _BAND, _BAND), sl],
                buf.at[slot, :, :, sl],
                sem_r.at[slot],
            ).wait()

        @pl.when(k == band)
        def _(k=k):
            for cp in _diag_quadrant_copies(x_ref, buf, sem_r, band, slot, k):
                cp.wait()


def _write_copy(o_ref, buf, sem_w, band, slot):
    return pltpu.make_async_copy(
        buf.at[slot],
        o_ref.at[:, pl.ds(band * _BAND, _BAND), :],
        sem_w.at[slot],
    )


def _body(x_ref, o_ref, buf, sem_r, sem_w):
    i = pl.program_id(0)
    n = pl.num_programs(0)
    s = x_ref.shape[2]
    nc = s // _BAND
    slot = jax.lax.rem(i, _SLOTS)
    nxt = jax.lax.rem(i + 1, _SLOTS)

    @pl.when(i == 0)
    def _():
        _read_band(x_ref, buf, sem_r, 0, jnp.int32(0), nc)

    @pl.when(jnp.logical_and(i + 1 < n, i >= _SLOTS - 1))
    def _():
        _write_copy(o_ref, buf, sem_w, i - (_SLOTS - 1), nxt).wait()

    @pl.when(i + 1 < n)
    def _():
        _read_band(x_ref, buf, sem_r, i + 1, nxt, nc)

    for k in range(nc):
        sl = slice(k * _BAND, (k + 1) * _BAND)

        @pl.when(k > i)
        def _(sl=sl):
            buf[slot, :, :, sl] = jnp.ones(
                (buf.shape[1], _BAND, _BAND), jnp.float32
            )

    _wait_band(x_ref, buf, sem_r, i, slot, nc)

    for k in range(nc):
        sl = slice(k * _BAND, (k + 1) * _BAND)

        @pl.when(k == i)
        def _(sl=sl):
            r = jax.lax.broadcasted_iota(jnp.int32, (1, _BAND, _BAND), 1)
            c = jax.lax.broadcasted_iota(jnp.int32, (1, _BAND, _BAND), 2)
            buf[slot, :, :, sl] = jnp.where(
                c > r, jnp.float32(1.0), buf[slot, :, :, sl]
            )

    _write_copy(o_ref, buf, sem_w, i, slot).start()

    n_static = x_ref.shape[1] // _BAND

    @pl.when(i == n - 1)
    def _():
        for band in range(max(0, n_static - _SLOTS), n_static):
            _write_copy(o_ref, buf, sem_w, band, band % _SLOTS).wait()


def kernel(x):
    batch, s, _ = x.shape
    n = s // _BAND
    return pl.pallas_call(
        _body,
        grid=(n,),
        in_specs=[pl.BlockSpec(memory_space=pltpu.MemorySpace.HBM)],
        out_specs=pl.BlockSpec(memory_space=pltpu.MemorySpace.HBM),
        out_shape=jax.ShapeDtypeStruct(x.shape, x.dtype),
        scratch_shapes=[
            pltpu.VMEM((_SLOTS, batch, _BAND, s), jnp.float32),
            pltpu.SemaphoreType.DMA((_SLOTS,)),
            pltpu.SemaphoreType.DMA((_SLOTS,)),
        ],
    )(x)


# speedup vs baseline: 2.8998x; 1.0071x over previous
import jax
import jax.numpy as jnp
from jax.experimental import pallas as pl
import jax.experimental.pallas.tpu as pltpu


_BAND = 256
_SLOTS = 6
_H = _BAND // 2


def _diag_squares(r0, c0, m):
    if m <= 128:
        return [(r0, c0, m, m)]
    h = m // 2
    return ([(r0 + h, c0, h, h)]
            + _diag_squares(r0, c0, h)
            + _diag_squares(r0 + h, c0 + h, h))


def _diag_quadrant_copies(x_ref, buf, sem_r, band, slot, k):
    base = k * _BAND
    out = []
    for r0, c0, hr, hc in _diag_squares(0, 0, _BAND):
        out.append(pltpu.make_async_copy(
            x_ref.at[:, pl.ds(band * _BAND + r0, hr),
                     slice(base + c0, base + c0 + hc)],
            buf.at[slot, :, slice(r0, r0 + hr),
                   slice(base + c0, base + c0 + hc)],
            sem_r.at[slot],
        ))
    return out


def _read_band(x_ref, buf, sem_r, band, slot, nc):
    for k in range(nc):
        sl = slice(k * _BAND, (k + 1) * _BAND)

        @pl.when(k < band)
        def _(sl=sl):
            pltpu.make_async_copy(
                x_ref.at[:, pl.ds(band * _BAND, _BAND), sl],
                buf.at[slot, :, :, sl],
                sem_r.at[slot],
            ).start()

        @pl.when(k == band)
        def _(k=k):
            for cp in _diag_quadrant_copies(x_ref, buf, sem_r, band, slot, k):
                cp.start()


def _wait_band(x_ref, buf, sem_r, band, slot, nc):
    for k in range(nc):
        sl = slice(k * _BAND, (k + 1) * _BAND)

        @pl.when(k < band)
        def _(sl=sl):
            pltpu.make_async_copy(
                x_ref.at[:, pl.ds(band * _BAND, _BAND), sl],
                buf.at[slot, :, :, sl],
                sem_r.at[slot],
            ).wait()

        @pl.when(k == band)
        def _(k=k):
            for cp in _diag_quadrant_copies(x_ref, buf, sem_r, band, slot, k):
                cp.wait()


def _write_copy(o_ref, buf, sem_w, band, slot):
    return pltpu.make_async_copy(
        buf.at[slot],
        o_ref.at[:, pl.ds(band * _BAND, _BAND), :],
        sem_w.at[slot],
    )


def _body(x_ref, o_ref, buf, sem_r, sem_w):
    i = pl.program_id(0)
    n = pl.num_programs(0)
    s = x_ref.shape[2]
    nc = s // _BAND
    slot = jax.lax.rem(i, _SLOTS)
    nxt = jax.lax.rem(i + 1, _SLOTS)

    @pl.when(i == 0)
    def _():
        _read_band(x_ref, buf, sem_r, 0, jnp.int32(0), nc)

    @pl.when(jnp.logical_and(i + 1 < n, i >= _SLOTS - 1))
    def _():
        _write_copy(o_ref, buf, sem_w, i - (_SLOTS - 1), nxt).wait()

    @pl.when(i + 1 < n)
    def _():
        _read_band(x_ref, buf, sem_r, i + 1, nxt, nc)

    for k in range(nc):
        sl = slice(k * _BAND, (k + 1) * _BAND)

        @pl.when(k > i)
        def _(sl=sl):
            buf[slot, :, :, sl] = jnp.ones(
                (buf.shape[1], _BAND, _BAND), jnp.float32
            )

    _wait_band(x_ref, buf, sem_r, i, slot, nc)

    for k in range(nc):
        sl = slice(k * _BAND, (k + 1) * _BAND)

        @pl.when(k == i)
        def _(sl=sl):
            r = jax.lax.broadcasted_iota(jnp.int32, (1, _BAND, _BAND), 1)
            c = jax.lax.broadcasted_iota(jnp.int32, (1, _BAND, _BAND), 2)
            buf[slot, :, :, sl] = jnp.where(
                c > r, jnp.float32(1.0), buf[slot, :, :, sl]
            )

    _write_copy(o_ref, buf, sem_w, i, slot).start()

    n_static = x_ref.shape[1] // _BAND

    @pl.when(i == n - 1)
    def _():
        for band in range(max(0, n_static - _SLOTS), n_static):
            _write_copy(o_ref, buf, sem_w, band, band % _SLOTS).wait()


def kernel(x):
    batch, s, _ = x.shape
    n = s // _BAND
    return pl.pallas_call(
        _body,
        grid=(n,),
        in_specs=[pl.BlockSpec(memory_space=pltpu.MemorySpace.HBM)],
        out_specs=pl.BlockSpec(memory_space=pltpu.MemorySpace.HBM),
        out_shape=jax.ShapeDtypeStruct(x.shape, x.dtype),
        scratch_shapes=[
            pltpu.VMEM((_SLOTS, batch, _BAND, s), jnp.float32),
            pltpu.SemaphoreType.DMA((_SLOTS,)),
            pltpu.SemaphoreType.DMA((_SLOTS,)),
        ],
    )(x)
